# Initial kernel scaffold; baseline (speedup 1.0000x reference)
import jax
import jax.numpy as jnp
from jax.experimental import pallas as pl


def kernel(image_tensor, keypoints, scale_weights):
    raise NotImplementedError("write your pallas kernel here")


# speedup vs baseline: 23.4135x; 23.4135x over previous
import jax
import jax.numpy as jnp
from jax.experimental import pallas as pl
from jax.experimental.pallas import tpu as pltpu

_SCALES = (1.0, 2.0, 4.0)
_PADS = tuple((int(6 * s) + 1) // 2 for s in _SCALES)
_NUM_KP = 15


def _plane_body(kp_ref, w_ref, out_ref):
    i = pl.program_id(0)
    b = i // _NUM_KP
    k = i % _NUM_KP
    x = kp_ref[b, k, 0]
    y = kp_ref[b, k, 1]
    H, W = out_ref.shape[2], out_ref.shape[3]

    dy = jax.lax.broadcasted_iota(jnp.int32, (H, 1), 0) - y
    dx = jax.lax.broadcasted_iota(jnp.int32, (1, W), 1) - x
    dy2 = (dy * dy).astype(jnp.float32)
    dx2 = (dx * dx).astype(jnp.float32)

    acc = jnp.zeros((H, W), jnp.float32)
    for s_idx, (sig, pad) in enumerate(zip(_SCALES, _PADS)):
        valid = (x >= pad) & (y >= pad) & (x < W - pad) & (y < H - pad)
        a = w_ref[s_idx] * valid.astype(jnp.float32)
        inv = 1.0 / (2.0 * sig * sig)
        fy = jnp.where(jnp.abs(dy) <= pad, jnp.exp(-dy2 * inv), 0.0)
        fx = jnp.where(jnp.abs(dx) <= pad, jnp.exp(-dx2 * inv), 0.0)
        acc = jnp.maximum(acc, a * (fy * fx))
    out_ref[0, 0, :, :] = acc


def kernel(image_tensor, keypoints, scale_weights):
    B, _, H, W = image_tensor.shape
    K = _NUM_KP
    kp = keypoints.astype(jnp.int32)
    grid = (B * K,)
    out = pl.pallas_call(
        _plane_body,
        grid=grid,
        in_specs=[
            pl.BlockSpec(memory_space=pltpu.SMEM),
            pl.BlockSpec(memory_space=pltpu.SMEM),
        ],
        out_specs=pl.BlockSpec(
            (1, 1, H, W), lambda i: (i // K, i % K, 0, 0)
        ),
        out_shape=jax.ShapeDtypeStruct((B, K, H, W), jnp.float32),
    )(kp, scale_weights)
    return out
